# baseline (speedup 1.0000x reference)
import functools

import jax
import jax.numpy as jnp
from jax import lax
from jax.experimental import pallas as pl
from jax.experimental.pallas import tpu as pltpu
from jax.experimental.pallas import tpu_sc as plsc

_R, _C = 128, 32768
_N = _R * _C
_POS = 0.75 * (_N - 1)
_K = int(_POS)
_FRAC = _POS - _K
_MIN32 = -2147483648
_MAX32 = 2147483647

_NC, _NS, _L = 2, 16, 16
_NW = _NC * _NS
_PER_W = _N // _NW
_CHUNK = 8192
_NCHUNKS = _PER_W // _CHUNK
_VPC = _CHUNK // _L
_BINS = 65536

_mesh = plsc.VectorSubcoreMesh(core_axis_name="c", subcore_axis_name="s")


def _keys_from(xv):
    bits = lax.bitcast_convert_type(xv, jnp.int32)
    u = jnp.where(bits >= 0, bits | jnp.int32(_MIN32), ~bits)
    return u ^ jnp.int32(_MIN32)


def _sc1_body(x_hbm, hist_out, sums_out, buf, hist, accbuf):
    wid = lax.axis_index("s") * _NC + lax.axis_index("c")
    base = wid * _PER_W

    def zero(i, carry):
        hist[pl.ds(i * _L, _L)] = jnp.zeros((_L,), jnp.int32)
        return carry

    lax.fori_loop(0, _BINS // _L, zero, 0)

    ones = jnp.ones((_L,), jnp.int32)

    def chunk(j, acc):
        pltpu.sync_copy(x_hbm.at[pl.ds(base + j * _CHUNK, _CHUNK)], buf)

        def vec(i, acc):
            xv = buf[pl.ds(i * _L, _L)]
            key = _keys_from(xv)
            hb = (key >> 16) + 32768
            plsc.addupdate_scatter(hist, [hb], ones)
            return acc + xv

        return lax.fori_loop(0, _VPC, vec, acc)

    acc = lax.fori_loop(0, _NCHUNKS, chunk, jnp.zeros((_L,), jnp.float32))
    accbuf[...] = acc
    pltpu.sync_copy(hist, hist_out.at[wid])
    pltpu.sync_copy(accbuf, sums_out.at[wid])


_sc_pass1 = functools.partial(
    pl.kernel,
    out_type=[
        jax.ShapeDtypeStruct((_NW, _BINS), jnp.int32),
        jax.ShapeDtypeStruct((_NW, _L), jnp.float32),
    ],
    mesh=_mesh,
    compiler_params=pltpu.CompilerParams(needs_layout_passes=False),
    scratch_types=[
        pltpu.VMEM((_CHUNK,), jnp.float32),
        pltpu.VMEM((_BINS,), jnp.int32),
        pltpu.VMEM((_L,), jnp.float32),
    ],
)(_sc1_body)


def _sc2_body(x_hbm, b1_hbm, hist_out, minab_out, buf, hist, b1buf, minbuf):
    wid = lax.axis_index("s") * _NC + lax.axis_index("c")
    base = wid * _PER_W

    def zero(i, carry):
        hist[pl.ds(i * _L, _L)] = jnp.zeros((_L,), jnp.int32)
        return carry

    lax.fori_loop(0, _BINS // _L, zero, 0)

    pltpu.sync_copy(b1_hbm, b1buf)
    b1v = b1buf[...]
    ones = jnp.ones((_L,), jnp.int32)

    def chunk(j, macc):
        pltpu.sync_copy(x_hbm.at[pl.ds(base + j * _CHUNK, _CHUNK)], buf)

        def vec(i, macc):
            xv = buf[pl.ds(i * _L, _L)]
            key = _keys_from(xv)
            hb = (key >> 16) + 32768
            low = key & 65535
            plsc.addupdate_scatter(hist, [low], ones, mask=hb == b1v)
            return jnp.minimum(
                macc, jnp.where(hb > b1v, key, jnp.int32(_MAX32)))

        return lax.fori_loop(0, _VPC, vec, macc)

    macc = lax.fori_loop(
        0, _NCHUNKS, chunk, jnp.full((_L,), _MAX32, jnp.int32))
    minbuf[...] = macc
    pltpu.sync_copy(hist, hist_out.at[wid])
    pltpu.sync_copy(minbuf, minab_out.at[wid])


_sc_pass2 = functools.partial(
    pl.kernel,
    out_type=[
        jax.ShapeDtypeStruct((_NW, _BINS), jnp.int32),
        jax.ShapeDtypeStruct((_NW, _L), jnp.int32),
    ],
    mesh=_mesh,
    compiler_params=pltpu.CompilerParams(needs_layout_passes=False),
    scratch_types=[
        pltpu.VMEM((_CHUNK,), jnp.float32),
        pltpu.VMEM((_BINS,), jnp.int32),
        pltpu.VMEM((_L,), jnp.int32),
        pltpu.VMEM((_L,), jnp.int32),
    ],
)(_sc2_body)


def _descend16(merged, rank):
    idx = (lax.broadcasted_iota(jnp.int32, merged.shape, 0) * 128
           + lax.broadcasted_iota(jnp.int32, merged.shape, 1))

    def step(i, p):
        t = p | (jnp.int32(1) << (jnp.int32(15) - i))
        cnt = jnp.sum(jnp.where(idx < t, merged, 0))
        return jnp.where(cnt > rank, p, t)

    return lax.fori_loop(0, 16, step, jnp.int32(0)), idx


def _an1_body(hist_ref, sums_ref, b1vec_ref, b1s_ref, below_ref, mean_ref):
    merged = jnp.sum(hist_ref[...], axis=0)
    b1, idx = _descend16(merged, jnp.int32(_K))
    below = jnp.sum(jnp.where(idx < b1, merged, 0))
    mean = jnp.sum(sums_ref[...]) / _N
    b1vec_ref[...] = jnp.full((_L,), b1, jnp.int32)
    b1s_ref[0, 0] = b1
    below_ref[0, 0] = below
    mean_ref[0, 0] = mean


def _tc_analyze1(hist, sums):
    return pl.pallas_call(
        _an1_body,
        in_specs=[
            pl.BlockSpec((_NW, 512, 128), lambda: (0, 0, 0)),
            pl.BlockSpec((_NW, _L), lambda: (0, 0)),
        ],
        out_specs=[
            pl.BlockSpec((_L,), lambda: (0,)),
            pl.BlockSpec(memory_space=pltpu.SMEM),
            pl.BlockSpec(memory_space=pltpu.SMEM),
            pl.BlockSpec(memory_space=pltpu.SMEM),
        ],
        out_shape=[
            jax.ShapeDtypeStruct((_L,), jnp.int32),
            jax.ShapeDtypeStruct((1, 1), jnp.int32),
            jax.ShapeDtypeStruct((1, 1), jnp.int32),
            jax.ShapeDtypeStruct((1, 1), jnp.float32),
        ],
    )(hist.reshape(_NW, 512, 128), sums)


def _key_to_float(k):
    u = k ^ jnp.int32(_MIN32)
    bits = jnp.where(u < 0, u & jnp.int32(_MAX32), ~u)
    return lax.bitcast_convert_type(bits, jnp.float32)


def _fin_body(x_ref, hist_ref, minab_ref, b1s_ref, below_ref, mean_ref,
              out_ref):
    b1 = b1s_ref[0, 0]
    below = below_ref[0, 0]
    m = mean_ref[0, 0]

    merged = jnp.sum(hist_ref[...], axis=0)
    cnt_b1 = jnp.sum(merged)
    r = jnp.int32(_K) - below

    low_k, _ = _descend16(merged, r)
    low_k1, _ = _descend16(merged, r + 1)
    min_above = jnp.min(minab_ref[...])

    hi_part = (b1 - 32768) << 16
    key_k = hi_part | low_k
    key_k1 = jnp.where(r + 1 < cnt_b1, hi_part | low_k1, min_above)

    xk = _key_to_float(key_k)
    xk1 = _key_to_float(key_k1)
    q = xk * (1.0 - _FRAC) + xk1 * _FRAC

    ch = 8

    def mask(ci, carry):
        xa = x_ref[pl.ds(ci * ch, ch), :]
        out_ref[pl.ds(ci * ch, ch), :] = jnp.where(xa > q, m, xa)
        return carry

    lax.fori_loop(0, _R // ch, mask, jnp.int32(0))


def _tc_finalize(image, hist2, minab, b1s, below, meanv):
    return pl.pallas_call(
        _fin_body,
        in_specs=[
            pl.BlockSpec((_R, _C), lambda: (0, 0)),
            pl.BlockSpec((_NW, 512, 128), lambda: (0, 0, 0)),
            pl.BlockSpec((_NW, _L), lambda: (0, 0)),
            pl.BlockSpec(memory_space=pltpu.SMEM),
            pl.BlockSpec(memory_space=pltpu.SMEM),
            pl.BlockSpec(memory_space=pltpu.SMEM),
        ],
        out_specs=pl.BlockSpec((_R, _C), lambda: (0, 0)),
        out_shape=jax.ShapeDtypeStruct((_R, _C), jnp.float32),
    )(image, hist2.reshape(_NW, 512, 128), minab, b1s, below, meanv)


@jax.jit
def kernel(image):
    x1d = image.reshape(_N)
    hist1, sums = _sc_pass1(x1d)
    b1vec, b1s, below, meanv = _tc_analyze1(hist1, sums)
    hist2, minab = _sc_pass2(x1d, b1vec)
    return _tc_finalize(image, hist2, minab, b1s, below, meanv)


# speedup vs baseline: 1.3674x; 1.3674x over previous
import functools

import jax
import jax.numpy as jnp
from jax import lax
from jax.experimental import pallas as pl
from jax.experimental.pallas import tpu as pltpu
from jax.experimental.pallas import tpu_sc as plsc

_R, _C = 128, 32768
_N = _R * _C
_POS = 0.75 * (_N - 1)
_K = int(_POS)
_FRAC = _POS - _K
_MIN32 = -2147483648
_MAX32 = 2147483647

_NC, _NS, _L = 2, 16, 16
_NW = _NC * _NS
_PER_W = _N // _NW
_CHUNK = 16384
_NCHUNKS = _PER_W // _CHUNK
_VPC = _CHUNK // _L
_BINS = 65536

_mesh = plsc.VectorSubcoreMesh(core_axis_name="c", subcore_axis_name="s")
_sc_params = pltpu.CompilerParams(needs_layout_passes=False)


def _upattern(xv):
    bits = lax.bitcast_convert_type(xv, jnp.int32)
    return jnp.where(bits >= 0, bits | jnp.int32(_MIN32), ~bits)


def _zero_hist(hist):
    def zero(i, carry):
        hist[pl.ds(i * _L, _L)] = jnp.zeros((_L,), jnp.int32)
        return carry

    lax.fori_loop(0, _BINS // _L, zero, 0, unroll=8)


def _sc1_body(x_hbm, hist_out, buf0, buf1, hist, sem0, sem1):
    wid = lax.axis_index("s") * _NC + lax.axis_index("c")
    base = wid * _PER_W
    bufs = (buf0, buf1)
    sems = (sem0, sem1)

    _zero_hist(hist)

    def dma(j, b):
        return pltpu.make_async_copy(
            x_hbm.at[pl.ds(base + j * _CHUNK, _CHUNK)], bufs[b], sems[b])

    dma(0, 0).start()
    dma(1, 1).start()

    ones = jnp.ones((_L,), jnp.int32)

    def process(buf):
        def vec(i, carry):
            xv = buf[pl.ds(i * _L, _L)]
            hb = lax.shift_right_logical(_upattern(xv), 16)
            plsc.addupdate_scatter(hist, [hb], ones)
            return carry

        lax.fori_loop(0, _VPC, vec, 0, unroll=8)

    def outer(j2, carry):
        for b in range(2):
            j = j2 * 2 + b
            dma(j, b).wait()
            process(bufs[b])

            @pl.when(j + 2 < _NCHUNKS)
            def _():
                dma(j + 2, b).start()

        return carry

    lax.fori_loop(0, _NCHUNKS // 2, outer, 0)
    pltpu.sync_copy(hist, hist_out.at[wid])


_sc_pass1 = functools.partial(
    pl.kernel,
    out_type=jax.ShapeDtypeStruct((_NW, _BINS), jnp.int32),
    mesh=_mesh,
    compiler_params=_sc_params,
    scratch_types=[
        pltpu.VMEM((_CHUNK,), jnp.float32),
        pltpu.VMEM((_CHUNK,), jnp.float32),
        pltpu.VMEM((_BINS,), jnp.int32),
        pltpu.SemaphoreType.DMA,
        pltpu.SemaphoreType.DMA,
    ],
)(_sc1_body)


def _sc2_body(x_hbm, b1_hbm, hist_out, minab_out,
              buf0, buf1, hist, b1buf, minbuf, sem0, sem1):
    wid = lax.axis_index("s") * _NC + lax.axis_index("c")
    base = wid * _PER_W
    bufs = (buf0, buf1)
    sems = (sem0, sem1)

    _zero_hist(hist)
    pltpu.sync_copy(b1_hbm, b1buf)
    b1v = b1buf[...]

    def dma(j, b):
        return pltpu.make_async_copy(
            x_hbm.at[pl.ds(base + j * _CHUNK, _CHUNK)], bufs[b], sems[b])

    dma(0, 0).start()
    dma(1, 1).start()

    ones = jnp.ones((_L,), jnp.int32)

    def process(buf, macc):
        def vec(i, macc):
            xv = buf[pl.ds(i * _L, _L)]
            u = _upattern(xv)
            hb = lax.shift_right_logical(u, 16)
            low = u & 65535
            plsc.addupdate_scatter(hist, [low], ones, mask=hb == b1v)
            key = u ^ jnp.int32(_MIN32)
            return jnp.minimum(
                macc, jnp.where(hb > b1v, key, jnp.int32(_MAX32)))

        return lax.fori_loop(0, _VPC, vec, macc, unroll=8)

    def outer(j2, macc):
        for b in range(2):
            j = j2 * 2 + b
            dma(j, b).wait()
            macc = process(bufs[b], macc)

            @pl.when(j + 2 < _NCHUNKS)
            def _():
                dma(j + 2, b).start()

        return macc

    macc = lax.fori_loop(
        0, _NCHUNKS // 2, outer, jnp.full((_L,), _MAX32, jnp.int32))
    minbuf[...] = macc
    pltpu.sync_copy(hist, hist_out.at[wid])
    pltpu.sync_copy(minbuf, minab_out.at[wid])


_sc_pass2 = functools.partial(
    pl.kernel,
    out_type=[
        jax.ShapeDtypeStruct((_NW, _BINS), jnp.int32),
        jax.ShapeDtypeStruct((_NW, _L), jnp.int32),
    ],
    mesh=_mesh,
    compiler_params=_sc_params,
    scratch_types=[
        pltpu.VMEM((_CHUNK,), jnp.float32),
        pltpu.VMEM((_CHUNK,), jnp.float32),
        pltpu.VMEM((_BINS,), jnp.int32),
        pltpu.VMEM((_L,), jnp.int32),
        pltpu.VMEM((_L,), jnp.int32),
        pltpu.SemaphoreType.DMA,
        pltpu.SemaphoreType.DMA,
    ],
)(_sc2_body)


def _descend16(merged, rank):
    idx = (lax.broadcasted_iota(jnp.int32, merged.shape, 0) * 128
           + lax.broadcasted_iota(jnp.int32, merged.shape, 1))

    def step(i, p):
        t = p | (jnp.int32(1) << (jnp.int32(15) - i))
        cnt = jnp.sum(jnp.where(idx < t, merged, 0))
        return jnp.where(cnt > rank, p, t)

    return lax.fori_loop(0, 16, step, jnp.int32(0)), idx


def _an1_body(hist_ref, x_ref, b1vec_ref, b1s_ref, below_ref, mean_ref):
    merged = jnp.sum(hist_ref[...], axis=0)
    b1, idx = _descend16(merged, jnp.int32(_K))
    below = jnp.sum(jnp.where(idx < b1, merged, 0))

    ch = 8

    def acc(ci, s):
        return s + jnp.sum(x_ref[pl.ds(ci * ch, ch), :])

    total = lax.fori_loop(0, _R // ch, acc, jnp.float32(0.0))

    b1vec_ref[...] = jnp.full((_L,), b1, jnp.int32)
    b1s_ref[0, 0] = b1
    below_ref[0, 0] = below
    mean_ref[0, 0] = total / _N


def _tc_analyze1(hist, image):
    return pl.pallas_call(
        _an1_body,
        in_specs=[
            pl.BlockSpec((_NW, 512, 128), lambda: (0, 0, 0)),
            pl.BlockSpec((_R, _C), lambda: (0, 0)),
        ],
        out_specs=[
            pl.BlockSpec((_L,), lambda: (0,)),
            pl.BlockSpec(memory_space=pltpu.SMEM),
            pl.BlockSpec(memory_space=pltpu.SMEM),
            pl.BlockSpec(memory_space=pltpu.SMEM),
        ],
        out_shape=[
            jax.ShapeDtypeStruct((_L,), jnp.int32),
            jax.ShapeDtypeStruct((1, 1), jnp.int32),
            jax.ShapeDtypeStruct((1, 1), jnp.int32),
            jax.ShapeDtypeStruct((1, 1), jnp.float32),
        ],
    )(hist.reshape(_NW, 512, 128), image)


def _key_to_float(k):
    u = k ^ jnp.int32(_MIN32)
    bits = jnp.where(u < 0, u & jnp.int32(_MAX32), ~u)
    return lax.bitcast_convert_type(bits, jnp.float32)


def _fin_body(x_ref, hist_ref, minab_ref, b1s_ref, below_ref, mean_ref,
              out_ref):
    b1 = b1s_ref[0, 0]
    below = below_ref[0, 0]
    m = mean_ref[0, 0]

    merged = jnp.sum(hist_ref[...], axis=0)
    cnt_b1 = jnp.sum(merged)
    r = jnp.int32(_K) - below

    low_k, _ = _descend16(merged, r)
    low_k1, _ = _descend16(merged, r + 1)
    min_above = jnp.min(minab_ref[...])

    hi_part = b1 << 16
    key_k = (hi_part | low_k) ^ jnp.int32(_MIN32)
    key_k1 = jnp.where(
        r + 1 < cnt_b1, (hi_part | low_k1) ^ jnp.int32(_MIN32), min_above)

    xk = _key_to_float(key_k)
    xk1 = _key_to_float(key_k1)
    q = xk * (1.0 - _FRAC) + xk1 * _FRAC

    ch = 8

    def mask(ci, carry):
        xa = x_ref[pl.ds(ci * ch, ch), :]
        out_ref[pl.ds(ci * ch, ch), :] = jnp.where(xa > q, m, xa)
        return carry

    lax.fori_loop(0, _R // ch, mask, jnp.int32(0))


def _tc_finalize(image, hist2, minab, b1s, below, meanv):
    return pl.pallas_call(
        _fin_body,
        in_specs=[
            pl.BlockSpec((_R, _C), lambda: (0, 0)),
            pl.BlockSpec((_NW, 512, 128), lambda: (0, 0, 0)),
            pl.BlockSpec((_NW, _L), lambda: (0, 0)),
            pl.BlockSpec(memory_space=pltpu.SMEM),
            pl.BlockSpec(memory_space=pltpu.SMEM),
            pl.BlockSpec(memory_space=pltpu.SMEM),
        ],
        out_specs=pl.BlockSpec((_R, _C), lambda: (0, 0)),
        out_shape=jax.ShapeDtypeStruct((_R, _C), jnp.float32),
    )(image, hist2.reshape(_NW, 512, 128), minab, b1s, below, meanv)


@jax.jit
def kernel(image):
    x1d = image.reshape(_N)
    hist1 = _sc_pass1(x1d)
    b1vec, b1s, below, meanv = _tc_analyze1(hist1, image)
    hist2, minab = _sc_pass2(x1d, b1vec)
    return _tc_finalize(image, hist2, minab, b1s, below, meanv)


# speedup vs baseline: 2.2612x; 1.6537x over previous
import functools

import jax
import jax.numpy as jnp
from jax import lax
from jax.experimental import pallas as pl
from jax.experimental.pallas import tpu as pltpu
from jax.experimental.pallas import tpu_sc as plsc

_R, _C = 128, 32768
_N = _R * _C
_POS = 0.75 * (_N - 1)
_K = int(_POS)
_FRAC = _POS - _K
_MIN32 = -2147483648
_MAX32 = 2147483647

_NC, _NS, _L = 2, 16, 16
_NW = _NC * _NS
_PER_W = _N // _NW
_CHUNK = 16384
_NCHUNKS = _PER_W // _CHUNK
_VPC = _CHUNK // _L
_BINS = 65536

_mesh = plsc.VectorSubcoreMesh(core_axis_name="c", subcore_axis_name="s")
_sc_params = pltpu.CompilerParams(needs_layout_passes=False)


def _upattern(xv):
    bits = lax.bitcast_convert_type(xv, jnp.int32)
    return jnp.where(bits >= 0, bits | jnp.int32(_MIN32), ~bits)


def _zero_hist(hist):
    def zero(i, carry):
        hist[pl.ds(i * _L, _L)] = jnp.zeros((_L,), jnp.int32)
        return carry

    lax.fori_loop(0, _BINS // _L, zero, 0, unroll=8)


def _sc1_body(x_hbm, hist_out, buf0, buf1, hist, sem0, sem1):
    wid = lax.axis_index("s") * _NC + lax.axis_index("c")
    base = wid * _PER_W
    bufs = (buf0, buf1)
    sems = (sem0, sem1)

    _zero_hist(hist)

    def dma(j, b):
        return pltpu.make_async_copy(
            x_hbm.at[pl.ds(base + j * _CHUNK, _CHUNK)], bufs[b], sems[b])

    dma(0, 0).start()
    dma(1, 1).start()

    ones = jnp.ones((_L,), jnp.int32)

    def process(buf):
        @plsc.parallel_loop(0, _VPC, step=1, unroll=8)
        def _(i):
            xv = buf[pl.ds(i * _L, _L)]
            hb = lax.shift_right_logical(_upattern(xv), 16)
            plsc.addupdate_scatter(hist, [hb], ones)

    def outer(j2, carry):
        for b in range(2):
            j = j2 * 2 + b
            dma(j, b).wait()
            process(bufs[b])

            @pl.when(j + 2 < _NCHUNKS)
            def _():
                dma(j + 2, b).start()

        return carry

    lax.fori_loop(0, _NCHUNKS // 2, outer, 0)
    pltpu.sync_copy(hist, hist_out.at[wid])


_sc_pass1 = functools.partial(
    pl.kernel,
    out_type=jax.ShapeDtypeStruct((_NW, _BINS), jnp.int32),
    mesh=_mesh,
    compiler_params=_sc_params,
    scratch_types=[
        pltpu.VMEM((_CHUNK,), jnp.float32),
        pltpu.VMEM((_CHUNK,), jnp.float32),
        pltpu.VMEM((_BINS,), jnp.int32),
        pltpu.SemaphoreType.DMA,
        pltpu.SemaphoreType.DMA,
    ],
)(_sc1_body)


def _sc2_body(x_hbm, b1_hbm, hist_out, minab_out,
              buf0, buf1, hist, b1buf, minbuf, sem0, sem1):
    wid = lax.axis_index("s") * _NC + lax.axis_index("c")
    base = wid * _PER_W
    bufs = (buf0, buf1)
    sems = (sem0, sem1)

    _zero_hist(hist)
    pltpu.sync_copy(b1_hbm, b1buf)
    b1v = b1buf[...]

    def dma(j, b):
        return pltpu.make_async_copy(
            x_hbm.at[pl.ds(base + j * _CHUNK, _CHUNK)], bufs[b], sems[b])

    dma(0, 0).start()
    dma(1, 1).start()

    ones = jnp.ones((_L,), jnp.int32)

    def process(buf, maccs):
        @plsc.parallel_loop(0, _VPC, step=1, unroll=8, carry=maccs)
        def vec(i, maccs):
            m0, m1 = maccs
            xv = buf[pl.ds(i * _L, _L)]
            u = _upattern(xv)
            hb = lax.shift_right_logical(u, 16)
            low = u & 65535
            plsc.addupdate_scatter(hist, [low], ones, mask=hb == b1v)
            key = u ^ jnp.int32(_MIN32)
            cand = jnp.where(hb > b1v, key, jnp.int32(_MAX32))
            return (jnp.minimum(m1, cand), m0)

        return vec

    def outer(j2, maccs):
        for b in range(2):
            j = j2 * 2 + b
            dma(j, b).wait()
            maccs = process(bufs[b], maccs)

            @pl.when(j + 2 < _NCHUNKS)
            def _():
                dma(j + 2, b).start()

        return maccs

    init = jnp.full((_L,), _MAX32, jnp.int32)
    m0, m1 = lax.fori_loop(0, _NCHUNKS // 2, outer, (init, init))
    minbuf[...] = jnp.minimum(m0, m1)
    pltpu.sync_copy(hist, hist_out.at[wid])
    pltpu.sync_copy(minbuf, minab_out.at[wid])


_sc_pass2 = functools.partial(
    pl.kernel,
    out_type=[
        jax.ShapeDtypeStruct((_NW, _BINS), jnp.int32),
        jax.ShapeDtypeStruct((_NW, _L), jnp.int32),
    ],
    mesh=_mesh,
    compiler_params=_sc_params,
    scratch_types=[
        pltpu.VMEM((_CHUNK,), jnp.float32),
        pltpu.VMEM((_CHUNK,), jnp.float32),
        pltpu.VMEM((_BINS,), jnp.int32),
        pltpu.VMEM((_L,), jnp.int32),
        pltpu.VMEM((_L,), jnp.int32),
        pltpu.SemaphoreType.DMA,
        pltpu.SemaphoreType.DMA,
    ],
)(_sc2_body)


def _descend16(merged, rank):
    idx = (lax.broadcasted_iota(jnp.int32, merged.shape, 0) * 128
           + lax.broadcasted_iota(jnp.int32, merged.shape, 1))

    def step(i, p):
        t = p | (jnp.int32(1) << (jnp.int32(15) - i))
        cnt = jnp.sum(jnp.where(idx < t, merged, 0))
        return jnp.where(cnt > rank, p, t)

    return lax.fori_loop(0, 16, step, jnp.int32(0)), idx


def _an1_body(hist_ref, x_ref, b1vec_ref, b1s_ref, below_ref, mean_ref):
    merged = jnp.sum(hist_ref[...], axis=0)
    b1, idx = _descend16(merged, jnp.int32(_K))
    below = jnp.sum(jnp.where(idx < b1, merged, 0))

    ch = 8

    def acc(ci, s):
        return s + jnp.sum(x_ref[pl.ds(ci * ch, ch), :])

    total = lax.fori_loop(0, _R // ch, acc, jnp.float32(0.0))

    b1vec_ref[...] = jnp.full((_L,), b1, jnp.int32)
    b1s_ref[0, 0] = b1
    below_ref[0, 0] = below
    mean_ref[0, 0] = total / _N


def _tc_analyze1(hist, image):
    return pl.pallas_call(
        _an1_body,
        in_specs=[
            pl.BlockSpec((_NW, 512, 128), lambda: (0, 0, 0)),
            pl.BlockSpec((_R, _C), lambda: (0, 0)),
        ],
        out_specs=[
            pl.BlockSpec((_L,), lambda: (0,)),
            pl.BlockSpec(memory_space=pltpu.SMEM),
            pl.BlockSpec(memory_space=pltpu.SMEM),
            pl.BlockSpec(memory_space=pltpu.SMEM),
        ],
        out_shape=[
            jax.ShapeDtypeStruct((_L,), jnp.int32),
            jax.ShapeDtypeStruct((1, 1), jnp.int32),
            jax.ShapeDtypeStruct((1, 1), jnp.int32),
            jax.ShapeDtypeStruct((1, 1), jnp.float32),
        ],
    )(hist.reshape(_NW, 512, 128), image)


def _key_to_float(k):
    u = k ^ jnp.int32(_MIN32)
    bits = jnp.where(u < 0, u & jnp.int32(_MAX32), ~u)
    return lax.bitcast_convert_type(bits, jnp.float32)


def _an2_body(hist_ref, minab_ref, b1s_ref, below_ref, q_ref):
    b1 = b1s_ref[0, 0]
    below = below_ref[0, 0]

    merged = jnp.sum(hist_ref[...], axis=0)
    cnt_b1 = jnp.sum(merged)
    r = jnp.int32(_K) - below

    low_k, _ = _descend16(merged, r)
    low_k1, _ = _descend16(merged, r + 1)
    min_above = jnp.min(minab_ref[...])

    hi_part = b1 << 16
    key_k = (hi_part | low_k) ^ jnp.int32(_MIN32)
    key_k1 = jnp.where(
        r + 1 < cnt_b1, (hi_part | low_k1) ^ jnp.int32(_MIN32), min_above)

    xk = _key_to_float(key_k)
    xk1 = _key_to_float(key_k1)
    q_ref[0, 0] = xk * (1.0 - _FRAC) + xk1 * _FRAC


def _tc_analyze2(hist2, minab, b1s, below):
    return pl.pallas_call(
        _an2_body,
        in_specs=[
            pl.BlockSpec((_NW, 512, 128), lambda: (0, 0, 0)),
            pl.BlockSpec((_NW, _L), lambda: (0, 0)),
            pl.BlockSpec(memory_space=pltpu.SMEM),
            pl.BlockSpec(memory_space=pltpu.SMEM),
        ],
        out_specs=pl.BlockSpec(memory_space=pltpu.SMEM),
        out_shape=jax.ShapeDtypeStruct((1, 1), jnp.float32),
    )(hist2.reshape(_NW, 512, 128), minab, b1s, below)


_MROWS = 8


def _mask_body(qs_ref, mean_ref, x_ref, out_ref):
    q = qs_ref[0, 0]
    m = mean_ref[0, 0]
    xa = x_ref[...]
    out_ref[...] = jnp.where(xa > q, m, xa)


def _tc_mask(image, qs, meanv):
    return pl.pallas_call(
        _mask_body,
        grid=(_R // _MROWS,),
        in_specs=[
            pl.BlockSpec(memory_space=pltpu.SMEM),
            pl.BlockSpec(memory_space=pltpu.SMEM),
            pl.BlockSpec((_MROWS, _C), lambda i: (i, 0)),
        ],
        out_specs=pl.BlockSpec((_MROWS, _C), lambda i: (i, 0)),
        out_shape=jax.ShapeDtypeStruct((_R, _C), jnp.float32),
    )(qs, meanv, image)


@jax.jit
def kernel(image):
    x1d = image.reshape(_N)
    hist1 = _sc_pass1(x1d)
    b1vec, b1s, below, meanv = _tc_analyze1(hist1, image)
    hist2, minab = _sc_pass2(x1d, b1vec)
    qs = _tc_analyze2(hist2, minab, b1s, below)
    return _tc_mask(image, qs, meanv)


# speedup vs baseline: 2.2642x; 1.0013x over previous
import functools

import jax
import jax.numpy as jnp
from jax import lax
from jax.experimental import pallas as pl
from jax.experimental.pallas import tpu as pltpu
from jax.experimental.pallas import tpu_sc as plsc

_R, _C = 128, 32768
_N = _R * _C
_POS = 0.75 * (_N - 1)
_K = int(_POS)
_FRAC = _POS - _K
_MIN32 = -2147483648
_MAX32 = 2147483647

_NC, _NS, _L = 2, 16, 16
_NW = _NC * _NS
_PER_W = _N // _NW
_CHUNK = 16384
_NCHUNKS = _PER_W // _CHUNK
_VPC = _CHUNK // _L
_BINS = 65536

_mesh = plsc.VectorSubcoreMesh(core_axis_name="c", subcore_axis_name="s")
_sc_params = pltpu.CompilerParams(
    needs_layout_passes=False, use_tc_tiling_on_sc=True)


def _upattern(xv):
    bits = lax.bitcast_convert_type(xv, jnp.int32)
    return jnp.where(bits >= 0, bits | jnp.int32(_MIN32), ~bits)


def _zero_hist(hist):
    def zero(i, carry):
        hist[pl.ds(i * _L, _L)] = jnp.zeros((_L,), jnp.int32)
        return carry

    lax.fori_loop(0, _BINS // _L, zero, 0, unroll=8)


def _sc1_body(x_hbm, hist_out, buf0, buf1, hist, sem0, sem1):
    wid = lax.axis_index("s") * _NC + lax.axis_index("c")
    base = wid * _PER_W
    bufs = (buf0, buf1)
    sems = (sem0, sem1)

    _zero_hist(hist)

    def dma(j, b):
        return pltpu.make_async_copy(
            x_hbm.at[pl.ds(base + j * _CHUNK, _CHUNK)], bufs[b], sems[b])

    dma(0, 0).start()
    dma(1, 1).start()

    ones = jnp.ones((_L,), jnp.int32)

    def process(buf):
        @plsc.parallel_loop(0, _VPC, step=1, unroll=8)
        def _(i):
            xv = buf[pl.ds(i * _L, _L)]
            hb = lax.shift_right_logical(_upattern(xv), 16)
            plsc.addupdate_scatter(hist, [hb], ones)

    def outer(j2, carry):
        for b in range(2):
            j = j2 * 2 + b
            dma(j, b).wait()
            process(bufs[b])

            @pl.when(j + 2 < _NCHUNKS)
            def _():
                dma(j + 2, b).start()

        return carry

    lax.fori_loop(0, _NCHUNKS // 2, outer, 0)
    pltpu.sync_copy(hist, hist_out.at[wid])


_sc_pass1 = functools.partial(
    pl.kernel,
    out_type=jax.ShapeDtypeStruct((_NW, _BINS), jnp.int32),
    mesh=_mesh,
    compiler_params=_sc_params,
    scratch_types=[
        pltpu.VMEM((_CHUNK,), jnp.float32),
        pltpu.VMEM((_CHUNK,), jnp.float32),
        pltpu.VMEM((_BINS,), jnp.int32),
        pltpu.SemaphoreType.DMA,
        pltpu.SemaphoreType.DMA,
    ],
)(_sc1_body)


def _sc2_body(x_hbm, b1_hbm, hist_out, minab_out,
              buf0, buf1, hist, b1buf, minbuf, sem0, sem1):
    wid = lax.axis_index("s") * _NC + lax.axis_index("c")
    base = wid * _PER_W
    bufs = (buf0, buf1)
    sems = (sem0, sem1)

    _zero_hist(hist)
    pltpu.sync_copy(b1_hbm, b1buf)
    b1v = b1buf[...]

    def dma(j, b):
        return pltpu.make_async_copy(
            x_hbm.at[pl.ds(base + j * _CHUNK, _CHUNK)], bufs[b], sems[b])

    dma(0, 0).start()
    dma(1, 1).start()

    ones = jnp.ones((_L,), jnp.int32)

    def process(buf, maccs):
        @plsc.parallel_loop(0, _VPC, step=1, unroll=8, carry=maccs)
        def vec(i, maccs):
            m0, m1 = maccs
            xv = buf[pl.ds(i * _L, _L)]
            u = _upattern(xv)
            hb = lax.shift_right_logical(u, 16)
            low = u & 65535
            plsc.addupdate_scatter(hist, [low], ones, mask=hb == b1v)
            key = u ^ jnp.int32(_MIN32)
            cand = jnp.where(hb > b1v, key, jnp.int32(_MAX32))
            return (jnp.minimum(m1, cand), m0)

        return vec

    def outer(j2, maccs):
        for b in range(2):
            j = j2 * 2 + b
            dma(j, b).wait()
            maccs = process(bufs[b], maccs)

            @pl.when(j + 2 < _NCHUNKS)
            def _():
                dma(j + 2, b).start()

        return maccs

    init = jnp.full((_L,), _MAX32, jnp.int32)
    m0, m1 = lax.fori_loop(0, _NCHUNKS // 2, outer, (init, init))
    minbuf[...] = jnp.minimum(m0, m1)
    pltpu.sync_copy(hist, hist_out.at[wid])
    pltpu.sync_copy(minbuf, minab_out.at[wid])


_sc_pass2 = functools.partial(
    pl.kernel,
    out_type=[
        jax.ShapeDtypeStruct((_NW, _BINS), jnp.int32),
        jax.ShapeDtypeStruct((_NW, _L), jnp.int32),
    ],
    mesh=_mesh,
    compiler_params=_sc_params,
    scratch_types=[
        pltpu.VMEM((_CHUNK,), jnp.float32),
        pltpu.VMEM((_CHUNK,), jnp.float32),
        pltpu.VMEM((_BINS,), jnp.int32),
        pltpu.VMEM((_L,), jnp.int32),
        pltpu.VMEM((_L,), jnp.int32),
        pltpu.SemaphoreType.DMA,
        pltpu.SemaphoreType.DMA,
    ],
)(_sc2_body)


def _descend16(merged, rank):
    idx = (lax.broadcasted_iota(jnp.int32, merged.shape, 0) * 128
           + lax.broadcasted_iota(jnp.int32, merged.shape, 1))

    def step(i, p):
        t = p | (jnp.int32(1) << (jnp.int32(15) - i))
        cnt = jnp.sum(jnp.where(idx < t, merged, 0))
        return jnp.where(cnt > rank, p, t)

    return lax.fori_loop(0, 16, step, jnp.int32(0)), idx


def _an1_body(hist_ref, x_ref, b1vec_ref, b1s_ref, below_ref, mean_ref):
    merged = jnp.sum(hist_ref[...], axis=0)
    b1, idx = _descend16(merged, jnp.int32(_K))
    below = jnp.sum(jnp.where(idx < b1, merged, 0))

    ch = 8

    def acc(ci, s):
        return s + jnp.sum(x_ref[pl.ds(ci * ch, ch), :])

    total = lax.fori_loop(0, _R // ch, acc, jnp.float32(0.0))

    b1vec_ref[...] = jnp.full((_L,), b1, jnp.int32)
    b1s_ref[0, 0] = b1
    below_ref[0, 0] = below
    mean_ref[0, 0] = total / _N


def _tc_analyze1(hist, image):
    return pl.pallas_call(
        _an1_body,
        in_specs=[
            pl.BlockSpec((_NW, 512, 128), lambda: (0, 0, 0)),
            pl.BlockSpec((_R, _C), lambda: (0, 0)),
        ],
        out_specs=[
            pl.BlockSpec((_L,), lambda: (0,)),
            pl.BlockSpec(memory_space=pltpu.SMEM),
            pl.BlockSpec(memory_space=pltpu.SMEM),
            pl.BlockSpec(memory_space=pltpu.SMEM),
        ],
        out_shape=[
            jax.ShapeDtypeStruct((_L,), jnp.int32),
            jax.ShapeDtypeStruct((1, 1), jnp.int32),
            jax.ShapeDtypeStruct((1, 1), jnp.int32),
            jax.ShapeDtypeStruct((1, 1), jnp.float32),
        ],
    )(hist.reshape(_NW, 512, 128), image)


def _key_to_float(k):
    u = k ^ jnp.int32(_MIN32)
    bits = jnp.where(u < 0, u & jnp.int32(_MAX32), ~u)
    return lax.bitcast_convert_type(bits, jnp.float32)


def _an2_body(hist_ref, minab_ref, b1s_ref, below_ref, q_ref):
    b1 = b1s_ref[0, 0]
    below = below_ref[0, 0]

    merged = jnp.sum(hist_ref[...], axis=0)
    cnt_b1 = jnp.sum(merged)
    r = jnp.int32(_K) - below

    low_k, _ = _descend16(merged, r)
    low_k1, _ = _descend16(merged, r + 1)
    min_above = jnp.min(minab_ref[...])

    hi_part = b1 << 16
    key_k = (hi_part | low_k) ^ jnp.int32(_MIN32)
    key_k1 = jnp.where(
        r + 1 < cnt_b1, (hi_part | low_k1) ^ jnp.int32(_MIN32), min_above)

    xk = _key_to_float(key_k)
    xk1 = _key_to_float(key_k1)
    q_ref[0, 0] = xk * (1.0 - _FRAC) + xk1 * _FRAC


def _tc_analyze2(hist2, minab, b1s, below):
    return pl.pallas_call(
        _an2_body,
        in_specs=[
            pl.BlockSpec((_NW, 512, 128), lambda: (0, 0, 0)),
            pl.BlockSpec((_NW, _L), lambda: (0, 0)),
            pl.BlockSpec(memory_space=pltpu.SMEM),
            pl.BlockSpec(memory_space=pltpu.SMEM),
        ],
        out_specs=pl.BlockSpec(memory_space=pltpu.SMEM),
        out_shape=jax.ShapeDtypeStruct((1, 1), jnp.float32),
    )(hist2.reshape(_NW, 512, 128), minab, b1s, below)


_MROWS = 8


def _mask_body(qs_ref, mean_ref, x_ref, out_ref):
    q = qs_ref[0, 0]
    m = mean_ref[0, 0]
    xa = x_ref[...]
    out_ref[...] = jnp.where(xa > q, m, xa)


def _tc_mask(image, qs, meanv):
    return pl.pallas_call(
        _mask_body,
        grid=(_R // _MROWS,),
        in_specs=[
            pl.BlockSpec(memory_space=pltpu.SMEM),
            pl.BlockSpec(memory_space=pltpu.SMEM),
            pl.BlockSpec((_MROWS, _C), lambda i: (i, 0)),
        ],
        out_specs=pl.BlockSpec((_MROWS, _C), lambda i: (i, 0)),
        out_shape=jax.ShapeDtypeStruct((_R, _C), jnp.float32),
    )(qs, meanv, image)


@jax.jit
def kernel(image):
    x1d = image.reshape(_N)
    hist1 = _sc_pass1(x1d)
    b1vec, b1s, below, meanv = _tc_analyze1(hist1, image)
    hist2, minab = _sc_pass2(x1d, b1vec)
    qs = _tc_analyze2(hist2, minab, b1s, below)
    return _tc_mask(image, qs, meanv)


# speedup vs baseline: 2.3994x; 1.0597x over previous
import functools

import jax
import jax.numpy as jnp
from jax import lax
from jax.experimental import pallas as pl
from jax.experimental.pallas import tpu as pltpu
from jax.experimental.pallas import tpu_sc as plsc

_R, _C = 128, 32768
_N = _R * _C
_POS = 0.75 * (_N - 1)
_K = int(_POS)
_FRAC = _POS - _K
_MIN32 = -2147483648
_MAX32 = 2147483647

_NC, _NS, _L = 2, 16, 16
_NW = _NC * _NS
_PER_W = _N // _NW
_CHUNK = 8192
_NCHUNKS = _PER_W // _CHUNK
_VPC = _CHUNK // _L
_BINS = 65536

_mesh = plsc.VectorSubcoreMesh(core_axis_name="c", subcore_axis_name="s")
_sc_params = pltpu.CompilerParams(needs_layout_passes=False)


def _upattern(xv):
    bits = lax.bitcast_convert_type(xv, jnp.int32)
    return jnp.where(bits >= 0, bits | jnp.int32(_MIN32), ~bits)


def _zero_hist(hist):
    def zero(i, carry):
        hist[pl.ds(i * _L, _L)] = jnp.zeros((_L,), jnp.int32)
        return carry

    lax.fori_loop(0, _BINS // _L, zero, 0, unroll=8)


_HALF = _BINS // 2
_SEG = _HALF // _NS


def _merge_hist(hist, shared, tmp, acc, hist_out, cid, sid):
    for h in range(2):
        off = h * _HALF
        pltpu.sync_copy(hist.at[pl.ds(off, _HALF)], shared.at[sid])
        plsc.subcore_barrier()
        seg = pl.ds(sid * _SEG, _SEG)
        pltpu.sync_copy(shared.at[0, seg], acc)

        def addt(t, carry):
            pltpu.sync_copy(shared.at[t, seg], tmp)

            @plsc.parallel_loop(0, _SEG // _L, step=1, unroll=8)
            def _(i):
                plsc.addupdate(
                    acc.at[pl.ds(i * _L, _L)], tmp[pl.ds(i * _L, _L)])

            return carry

        lax.fori_loop(1, _NS, addt, 0)
        pltpu.sync_copy(
            acc, hist_out.at[cid, pl.ds(off + sid * _SEG, _SEG)])
        plsc.subcore_barrier()


def _sc1_body(x_hbm, hist_out, sums_out,
              buf0, buf1, hist, sumbuf, shared, tmp, acc, sem0, sem1):
    cid = lax.axis_index("c")
    sid = lax.axis_index("s")
    wid = sid * _NC + cid
    base = wid * _PER_W
    bufs = (buf0, buf1)
    sems = (sem0, sem1)

    _zero_hist(hist)

    def dma(j, b):
        return pltpu.make_async_copy(
            x_hbm.at[pl.ds(base + j * _CHUNK, _CHUNK)], bufs[b], sems[b])

    dma(0, 0).start()
    dma(1, 1).start()

    ones = jnp.ones((_L,), jnp.int32)

    def process(buf, sums):
        @plsc.parallel_loop(0, _VPC, step=1, unroll=8, carry=sums)
        def vec(i, sums):
            s0, s1 = sums
            xv = buf[pl.ds(i * _L, _L)]
            hb = lax.shift_right_logical(_upattern(xv), 16)
            plsc.addupdate_scatter(hist, [hb], ones)
            return (s1 + xv, s0)

        return vec

    def outer(j2, sums):
        for b in range(2):
            j = j2 * 2 + b
            dma(j, b).wait()
            sums = process(bufs[b], sums)

            @pl.when(j + 2 < _NCHUNKS)
            def _():
                dma(j + 2, b).start()

        return sums

    zero = jnp.zeros((_L,), jnp.float32)
    s0, s1 = lax.fori_loop(0, _NCHUNKS // 2, outer, (zero, zero))
    sumbuf[...] = s0 + s1
    pltpu.sync_copy(sumbuf, sums_out.at[wid])
    _merge_hist(hist, shared, tmp, acc, hist_out, cid, sid)


_sc_pass1 = functools.partial(
    pl.kernel,
    out_type=[
        jax.ShapeDtypeStruct((_NC, _BINS), jnp.int32),
        jax.ShapeDtypeStruct((_NW, _L), jnp.float32),
    ],
    mesh=_mesh,
    compiler_params=_sc_params,
    scratch_types=[
        pltpu.VMEM((_CHUNK,), jnp.float32),
        pltpu.VMEM((_CHUNK,), jnp.float32),
        pltpu.VMEM((_BINS,), jnp.int32),
        pltpu.VMEM((_L,), jnp.float32),
        pltpu.VMEM_SHARED((_NS, _HALF), jnp.int32),
        pltpu.VMEM((_SEG,), jnp.int32),
        pltpu.VMEM((_SEG,), jnp.int32),
        pltpu.SemaphoreType.DMA,
        pltpu.SemaphoreType.DMA,
    ],
)(_sc1_body)


def _sc2_body(x_hbm, b1_hbm, hist_out, minab_out,
              buf0, buf1, hist, b1buf, minbuf, shared, tmp, acc, sem0, sem1):
    cid = lax.axis_index("c")
    sid = lax.axis_index("s")
    wid = sid * _NC + cid
    base = wid * _PER_W
    bufs = (buf0, buf1)
    sems = (sem0, sem1)

    _zero_hist(hist)
    pltpu.sync_copy(b1_hbm, b1buf)
    b1v = b1buf[...]

    def dma(j, b):
        return pltpu.make_async_copy(
            x_hbm.at[pl.ds(base + j * _CHUNK, _CHUNK)], bufs[b], sems[b])

    dma(0, 0).start()
    dma(1, 1).start()

    ones = jnp.ones((_L,), jnp.int32)

    def process(buf, maccs):
        @plsc.parallel_loop(0, _VPC, step=1, unroll=8, carry=maccs)
        def vec(i, maccs):
            m0, m1 = maccs
            xv = buf[pl.ds(i * _L, _L)]
            u = _upattern(xv)
            hb = lax.shift_right_logical(u, 16)
            low = u & 65535
            plsc.addupdate_scatter(hist, [low], ones, mask=hb == b1v)
            key = u ^ jnp.int32(_MIN32)
            cand = jnp.where(hb > b1v, key, jnp.int32(_MAX32))
            return (jnp.minimum(m1, cand), m0)

        return vec

    def outer(j2, maccs):
        for b in range(2):
            j = j2 * 2 + b
            dma(j, b).wait()
            maccs = process(bufs[b], maccs)

            @pl.when(j + 2 < _NCHUNKS)
            def _():
                dma(j + 2, b).start()

        return maccs

    init = jnp.full((_L,), _MAX32, jnp.int32)
    m0, m1 = lax.fori_loop(0, _NCHUNKS // 2, outer, (init, init))
    minbuf[...] = jnp.minimum(m0, m1)
    pltpu.sync_copy(minbuf, minab_out.at[wid])
    _merge_hist(hist, shared, tmp, acc, hist_out, cid, sid)


_sc_pass2 = functools.partial(
    pl.kernel,
    out_type=[
        jax.ShapeDtypeStruct((_NC, _BINS), jnp.int32),
        jax.ShapeDtypeStruct((_NW, _L), jnp.int32),
    ],
    mesh=_mesh,
    compiler_params=_sc_params,
    scratch_types=[
        pltpu.VMEM((_CHUNK,), jnp.float32),
        pltpu.VMEM((_CHUNK,), jnp.float32),
        pltpu.VMEM((_BINS,), jnp.int32),
        pltpu.VMEM((_L,), jnp.int32),
        pltpu.VMEM((_L,), jnp.int32),
        pltpu.VMEM_SHARED((_NS, _HALF), jnp.int32),
        pltpu.VMEM((_SEG,), jnp.int32),
        pltpu.VMEM((_SEG,), jnp.int32),
        pltpu.SemaphoreType.DMA,
        pltpu.SemaphoreType.DMA,
    ],
)(_sc2_body)


def _descend16(merged, rank):
    idx = (lax.broadcasted_iota(jnp.int32, merged.shape, 0) * 128
           + lax.broadcasted_iota(jnp.int32, merged.shape, 1))

    def step(i, p):
        t = p | (jnp.int32(1) << (jnp.int32(15) - i))
        cnt = jnp.sum(jnp.where(idx < t, merged, 0))
        return jnp.where(cnt > rank, p, t)

    return lax.fori_loop(0, 16, step, jnp.int32(0)), idx


def _an1_body(hist_ref, sums_ref, b1vec_ref, b1s_ref, below_ref, mean_ref):
    merged = hist_ref[0] + hist_ref[1]
    b1, idx = _descend16(merged, jnp.int32(_K))
    below = jnp.sum(jnp.where(idx < b1, merged, 0))

    b1vec_ref[...] = jnp.full((_L,), b1, jnp.int32)
    b1s_ref[0, 0] = b1
    below_ref[0, 0] = below
    mean_ref[0, 0] = jnp.sum(sums_ref[...]) / _N


def _tc_analyze1(hist, sums):
    return pl.pallas_call(
        _an1_body,
        in_specs=[
            pl.BlockSpec((_NC, 512, 128), lambda: (0, 0, 0)),
            pl.BlockSpec((_NW, _L), lambda: (0, 0)),
        ],
        out_specs=[
            pl.BlockSpec((_L,), lambda: (0,)),
            pl.BlockSpec(memory_space=pltpu.SMEM),
            pl.BlockSpec(memory_space=pltpu.SMEM),
            pl.BlockSpec(memory_space=pltpu.SMEM),
        ],
        out_shape=[
            jax.ShapeDtypeStruct((_L,), jnp.int32),
            jax.ShapeDtypeStruct((1, 1), jnp.int32),
            jax.ShapeDtypeStruct((1, 1), jnp.int32),
            jax.ShapeDtypeStruct((1, 1), jnp.float32),
        ],
    )(hist.reshape(_NC, 512, 128), sums)


def _key_to_float(k):
    u = k ^ jnp.int32(_MIN32)
    bits = jnp.where(u < 0, u & jnp.int32(_MAX32), ~u)
    return lax.bitcast_convert_type(bits, jnp.float32)


def _an2_body(hist_ref, minab_ref, b1s_ref, below_ref, q_ref):
    b1 = b1s_ref[0, 0]
    below = below_ref[0, 0]

    merged = hist_ref[0] + hist_ref[1]
    cnt_b1 = jnp.sum(merged)
    r = jnp.int32(_K) - below

    low_k, _ = _descend16(merged, r)
    low_k1, _ = _descend16(merged, r + 1)
    min_above = jnp.min(minab_ref[...])

    hi_part = b1 << 16
    key_k = (hi_part | low_k) ^ jnp.int32(_MIN32)
    key_k1 = jnp.where(
        r + 1 < cnt_b1, (hi_part | low_k1) ^ jnp.int32(_MIN32), min_above)

    xk = _key_to_float(key_k)
    xk1 = _key_to_float(key_k1)
    q_ref[0, 0] = xk * (1.0 - _FRAC) + xk1 * _FRAC


def _tc_analyze2(hist2, minab, b1s, below):
    return pl.pallas_call(
        _an2_body,
        in_specs=[
            pl.BlockSpec((_NC, 512, 128), lambda: (0, 0, 0)),
            pl.BlockSpec((_NW, _L), lambda: (0, 0)),
            pl.BlockSpec(memory_space=pltpu.SMEM),
            pl.BlockSpec(memory_space=pltpu.SMEM),
        ],
        out_specs=pl.BlockSpec(memory_space=pltpu.SMEM),
        out_shape=jax.ShapeDtypeStruct((1, 1), jnp.float32),
    )(hist2.reshape(_NC, 512, 128), minab, b1s, below)


_MROWS = 8


def _mask_body(qs_ref, mean_ref, x_ref, out_ref):
    q = qs_ref[0, 0]
    m = mean_ref[0, 0]
    xa = x_ref[...]
    out_ref[...] = jnp.where(xa > q, m, xa)


def _tc_mask(image, qs, meanv):
    return pl.pallas_call(
        _mask_body,
        grid=(_R // _MROWS,),
        in_specs=[
            pl.BlockSpec(memory_space=pltpu.SMEM),
            pl.BlockSpec(memory_space=pltpu.SMEM),
            pl.BlockSpec((_MROWS, _C), lambda i: (i, 0)),
        ],
        out_specs=pl.BlockSpec((_MROWS, _C), lambda i: (i, 0)),
        out_shape=jax.ShapeDtypeStruct((_R, _C), jnp.float32),
    )(qs, meanv, image)


@jax.jit
def kernel(image):
    x1d = image.reshape(_N)
    hist1, sums = _sc_pass1(x1d)
    b1vec, b1s, below, meanv = _tc_analyze1(hist1, sums)
    hist2, minab = _sc_pass2(x1d, b1vec)
    qs = _tc_analyze2(hist2, minab, b1s, below)
    return _tc_mask(image, qs, meanv)


# speedup vs baseline: 2.4661x; 1.0278x over previous
import functools

import jax
import jax.numpy as jnp
from jax import lax
from jax.experimental import pallas as pl
from jax.experimental.pallas import tpu as pltpu
from jax.experimental.pallas import tpu_sc as plsc

_R, _C = 128, 32768
_N = _R * _C
_POS = 0.75 * (_N - 1)
_K = int(_POS)
_FRAC = _POS - _K
_MIN32 = -2147483648
_MAX32 = 2147483647

_NC, _NS, _L = 2, 16, 16
_NW = _NC * _NS
_PER_W = _N // _NW
_CHUNK = 4096
_NCHUNKS = _PER_W // _CHUNK
_VPC = _CHUNK // _L
_BINS = 65536

_mesh = plsc.VectorSubcoreMesh(core_axis_name="c", subcore_axis_name="s")
_sc_params = pltpu.CompilerParams(needs_layout_passes=False)


def _upattern(xv):
    bits = lax.bitcast_convert_type(xv, jnp.int32)
    return jnp.where(bits >= 0, bits | jnp.int32(_MIN32), ~bits)


def _zero_hist(hist):
    def zero(i, carry):
        hist[pl.ds(i * _L, _L)] = jnp.zeros((_L,), jnp.int32)
        return carry

    lax.fori_loop(0, _BINS // _L, zero, 0, unroll=8)


_HALF = _BINS // 2
_SEG = _HALF // _NS


def _merge_hist(hist, shared, seg2d, accbuf, hist_out, cid, sid):
    for h in range(2):
        off = h * _HALF
        pltpu.sync_copy(hist.at[pl.ds(off, _HALF)], shared.at[sid])
        plsc.subcore_barrier()
        for g in range(2):
            pltpu.sync_copy(
                shared.at[pl.ds(g * 8, 8), pl.ds(sid * _SEG, _SEG)], seg2d)

            @plsc.parallel_loop(0, _SEG // _L, step=1, unroll=4)
            def _(i):
                sl = pl.ds(i * _L, _L)
                v = seg2d[0, sl]
                for t in range(1, 8):
                    v = v + seg2d[t, sl]
                if g == 0:
                    accbuf[sl] = v
                else:
                    accbuf[sl] = accbuf[sl] + v

        pltpu.sync_copy(
            accbuf, hist_out.at[cid, pl.ds(off + sid * _SEG, _SEG)])
        plsc.subcore_barrier()


def _sc1_body(x_hbm, hist_out, sums_out,
              buf0, buf1, hist, sumbuf, shared, seg2d, accbuf, sem0, sem1):
    cid = lax.axis_index("c")
    sid = lax.axis_index("s")
    wid = sid * _NC + cid
    base = wid * _PER_W
    bufs = (buf0, buf1)
    sems = (sem0, sem1)

    _zero_hist(hist)

    def dma(j, b):
        return pltpu.make_async_copy(
            x_hbm.at[pl.ds(base + j * _CHUNK, _CHUNK)], bufs[b], sems[b])

    dma(0, 0).start()
    dma(1, 1).start()

    ones = jnp.ones((_L,), jnp.int32)

    def process(buf, sums):
        @plsc.parallel_loop(0, _VPC, step=1, unroll=8, carry=sums)
        def vec(i, sums):
            s0, s1 = sums
            xv = buf[pl.ds(i * _L, _L)]
            hb = lax.shift_right_logical(_upattern(xv), 16)
            plsc.addupdate_scatter(hist, [hb], ones)
            return (s1 + xv, s0)

        return vec

    def outer(j2, sums):
        for b in range(2):
            j = j2 * 2 + b
            dma(j, b).wait()
            sums = process(bufs[b], sums)

            @pl.when(j + 2 < _NCHUNKS)
            def _():
                dma(j + 2, b).start()

        return sums

    zero = jnp.zeros((_L,), jnp.float32)
    s0, s1 = lax.fori_loop(0, _NCHUNKS // 2, outer, (zero, zero))
    sumbuf[...] = s0 + s1
    pltpu.sync_copy(sumbuf, sums_out.at[wid])
    _merge_hist(hist, shared, seg2d, accbuf, hist_out, cid, sid)


_sc_pass1 = functools.partial(
    pl.kernel,
    out_type=[
        jax.ShapeDtypeStruct((_NC, _BINS), jnp.int32),
        jax.ShapeDtypeStruct((_NW, _L), jnp.float32),
    ],
    mesh=_mesh,
    compiler_params=_sc_params,
    scratch_types=[
        pltpu.VMEM((_CHUNK,), jnp.float32),
        pltpu.VMEM((_CHUNK,), jnp.float32),
        pltpu.VMEM((_BINS,), jnp.int32),
        pltpu.VMEM((_L,), jnp.float32),
        pltpu.VMEM_SHARED((_NS, _HALF), jnp.int32),
        pltpu.VMEM((8, _SEG), jnp.int32),
        pltpu.VMEM((_SEG,), jnp.int32),
        pltpu.SemaphoreType.DMA,
        pltpu.SemaphoreType.DMA,
    ],
)(_sc1_body)


def _sc2_body(x_hbm, b1_hbm, hist_out, minab_out,
              buf0, buf1, hist, b1buf, minbuf, shared, seg2d, accbuf,
              sem0, sem1):
    cid = lax.axis_index("c")
    sid = lax.axis_index("s")
    wid = sid * _NC + cid
    base = wid * _PER_W
    bufs = (buf0, buf1)
    sems = (sem0, sem1)

    _zero_hist(hist)
    pltpu.sync_copy(b1_hbm, b1buf)
    b1v = b1buf[...]

    def dma(j, b):
        return pltpu.make_async_copy(
            x_hbm.at[pl.ds(base + j * _CHUNK, _CHUNK)], bufs[b], sems[b])

    dma(0, 0).start()
    dma(1, 1).start()

    ones = jnp.ones((_L,), jnp.int32)

    def process(buf, maccs):
        @plsc.parallel_loop(0, _VPC, step=1, unroll=8, carry=maccs)
        def vec(i, maccs):
            m0, m1 = maccs
            xv = buf[pl.ds(i * _L, _L)]
            u = _upattern(xv)
            hb = lax.shift_right_logical(u, 16)
            low = u & 65535
            plsc.addupdate_scatter(hist, [low], ones, mask=hb == b1v)
            key = u ^ jnp.int32(_MIN32)
            cand = jnp.where(hb > b1v, key, jnp.int32(_MAX32))
            return (jnp.minimum(m1, cand), m0)

        return vec

    def outer(j2, maccs):
        for b in range(2):
            j = j2 * 2 + b
            dma(j, b).wait()
            maccs = process(bufs[b], maccs)

            @pl.when(j + 2 < _NCHUNKS)
            def _():
                dma(j + 2, b).start()

        return maccs

    init = jnp.full((_L,), _MAX32, jnp.int32)
    m0, m1 = lax.fori_loop(0, _NCHUNKS // 2, outer, (init, init))
    minbuf[...] = jnp.minimum(m0, m1)
    pltpu.sync_copy(minbuf, minab_out.at[wid])
    _merge_hist(hist, shared, seg2d, accbuf, hist_out, cid, sid)


_sc_pass2 = functools.partial(
    pl.kernel,
    out_type=[
        jax.ShapeDtypeStruct((_NC, _BINS), jnp.int32),
        jax.ShapeDtypeStruct((_NW, _L), jnp.int32),
    ],
    mesh=_mesh,
    compiler_params=_sc_params,
    scratch_types=[
        pltpu.VMEM((_CHUNK,), jnp.float32),
        pltpu.VMEM((_CHUNK,), jnp.float32),
        pltpu.VMEM((_BINS,), jnp.int32),
        pltpu.VMEM((_L,), jnp.int32),
        pltpu.VMEM((_L,), jnp.int32),
        pltpu.VMEM_SHARED((_NS, _HALF), jnp.int32),
        pltpu.VMEM((8, _SEG), jnp.int32),
        pltpu.VMEM((_SEG,), jnp.int32),
        pltpu.SemaphoreType.DMA,
        pltpu.SemaphoreType.DMA,
    ],
)(_sc2_body)


def _descend16(merged, rank):
    idx = (lax.broadcasted_iota(jnp.int32, merged.shape, 0) * 128
           + lax.broadcasted_iota(jnp.int32, merged.shape, 1))

    def step(i, p):
        t = p | (jnp.int32(1) << (jnp.int32(15) - i))
        cnt = jnp.sum(jnp.where(idx < t, merged, 0))
        return jnp.where(cnt > rank, p, t)

    return lax.fori_loop(0, 16, step, jnp.int32(0)), idx


def _an1_body(hist_ref, sums_ref, b1vec_ref, b1s_ref, below_ref, mean_ref):
    merged = hist_ref[0] + hist_ref[1]
    b1, idx = _descend16(merged, jnp.int32(_K))
    below = jnp.sum(jnp.where(idx < b1, merged, 0))

    b1vec_ref[...] = jnp.full((_L,), b1, jnp.int32)
    b1s_ref[0, 0] = b1
    below_ref[0, 0] = below
    mean_ref[0, 0] = jnp.sum(sums_ref[...]) / _N


def _tc_analyze1(hist, sums):
    return pl.pallas_call(
        _an1_body,
        in_specs=[
            pl.BlockSpec((_NC, 512, 128), lambda: (0, 0, 0)),
            pl.BlockSpec((_NW, _L), lambda: (0, 0)),
        ],
        out_specs=[
            pl.BlockSpec((_L,), lambda: (0,)),
            pl.BlockSpec(memory_space=pltpu.SMEM),
            pl.BlockSpec(memory_space=pltpu.SMEM),
            pl.BlockSpec(memory_space=pltpu.SMEM),
        ],
        out_shape=[
            jax.ShapeDtypeStruct((_L,), jnp.int32),
            jax.ShapeDtypeStruct((1, 1), jnp.int32),
            jax.ShapeDtypeStruct((1, 1), jnp.int32),
            jax.ShapeDtypeStruct((1, 1), jnp.float32),
        ],
    )(hist.reshape(_NC, 512, 128), sums)


def _key_to_float(k):
    u = k ^ jnp.int32(_MIN32)
    bits = jnp.where(u < 0, u & jnp.int32(_MAX32), ~u)
    return lax.bitcast_convert_type(bits, jnp.float32)


def _an2_body(hist_ref, minab_ref, b1s_ref, below_ref, q_ref):
    b1 = b1s_ref[0, 0]
    below = below_ref[0, 0]

    merged = hist_ref[0] + hist_ref[1]
    cnt_b1 = jnp.sum(merged)
    r = jnp.int32(_K) - below

    low_k, _ = _descend16(merged, r)
    low_k1, _ = _descend16(merged, r + 1)
    min_above = jnp.min(minab_ref[...])

    hi_part = b1 << 16
    key_k = (hi_part | low_k) ^ jnp.int32(_MIN32)
    key_k1 = jnp.where(
        r + 1 < cnt_b1, (hi_part | low_k1) ^ jnp.int32(_MIN32), min_above)

    xk = _key_to_float(key_k)
    xk1 = _key_to_float(key_k1)
    q_ref[0, 0] = xk * (1.0 - _FRAC) + xk1 * _FRAC


def _tc_analyze2(hist2, minab, b1s, below):
    return pl.pallas_call(
        _an2_body,
        in_specs=[
            pl.BlockSpec((_NC, 512, 128), lambda: (0, 0, 0)),
            pl.BlockSpec((_NW, _L), lambda: (0, 0)),
            pl.BlockSpec(memory_space=pltpu.SMEM),
            pl.BlockSpec(memory_space=pltpu.SMEM),
        ],
        out_specs=pl.BlockSpec(memory_space=pltpu.SMEM),
        out_shape=jax.ShapeDtypeStruct((1, 1), jnp.float32),
    )(hist2.reshape(_NC, 512, 128), minab, b1s, below)


_MROWS = 8


def _mask_body(qs_ref, mean_ref, x_ref, out_ref):
    q = qs_ref[0, 0]
    m = mean_ref[0, 0]
    xa = x_ref[...]
    out_ref[...] = jnp.where(xa > q, m, xa)


def _tc_mask(image, qs, meanv):
    return pl.pallas_call(
        _mask_body,
        grid=(_R // _MROWS,),
        in_specs=[
            pl.BlockSpec(memory_space=pltpu.SMEM),
            pl.BlockSpec(memory_space=pltpu.SMEM),
            pl.BlockSpec((_MROWS, _C), lambda i: (i, 0)),
        ],
        out_specs=pl.BlockSpec((_MROWS, _C), lambda i: (i, 0)),
        out_shape=jax.ShapeDtypeStruct((_R, _C), jnp.float32),
    )(qs, meanv, image)


@jax.jit
def kernel(image):
    x1d = image.reshape(_N)
    hist1, sums = _sc_pass1(x1d)
    b1vec, b1s, below, meanv = _tc_analyze1(hist1, sums)
    hist2, minab = _sc_pass2(x1d, b1vec)
    qs = _tc_analyze2(hist2, minab, b1s, below)
    return _tc_mask(image, qs, meanv)


# speedup vs baseline: 2.5079x; 1.0169x over previous
import functools

import jax
import jax.numpy as jnp
from jax import lax
from jax.experimental import pallas as pl
from jax.experimental.pallas import tpu as pltpu
from jax.experimental.pallas import tpu_sc as plsc

_R, _C = 128, 32768
_N = _R * _C
_POS = 0.75 * (_N - 1)
_K = int(_POS)
_FRAC = _POS - _K
_MIN32 = -2147483648
_MAX32 = 2147483647

_NC, _NS, _L = 2, 16, 16
_NW = _NC * _NS
_PER_W = _N // _NW
_CHUNK = 4096
_NCHUNKS = _PER_W // _CHUNK
_VPC = _CHUNK // _L
_BINS = 65536

_mesh = plsc.VectorSubcoreMesh(core_axis_name="c", subcore_axis_name="s")
_sc_params = pltpu.CompilerParams(needs_layout_passes=False)


def _upattern(xv):
    bits = lax.bitcast_convert_type(xv, jnp.int32)
    return jnp.where(bits >= 0, bits | jnp.int32(_MIN32), ~bits)


def _zero_hist(hist):
    def zero(i, carry):
        hist[pl.ds(i * _L, _L)] = jnp.zeros((_L,), jnp.int32)
        return carry

    lax.fori_loop(0, _BINS // _L, zero, 0, unroll=8)


_HALF = _BINS // 2
_SEG = _HALF // _NS


def _merge_hist(hist, shared, seg2d, accbuf, hist_out, cid, sid):
    for h in range(2):
        off = h * _HALF
        pltpu.sync_copy(hist.at[pl.ds(off, _HALF)], shared.at[sid])
        plsc.subcore_barrier()
        for g in range(2):
            pltpu.sync_copy(
                shared.at[pl.ds(g * 8, 8), pl.ds(sid * _SEG, _SEG)], seg2d)

            @plsc.parallel_loop(0, _SEG // _L, step=1, unroll=4)
            def _(i):
                sl = pl.ds(i * _L, _L)
                v = seg2d[0, sl]
                for t in range(1, 8):
                    v = v + seg2d[t, sl]
                if g == 0:
                    accbuf[sl] = v
                else:
                    accbuf[sl] = accbuf[sl] + v

        pltpu.sync_copy(
            accbuf, hist_out.at[cid, pl.ds(off + sid * _SEG, _SEG)])
        plsc.subcore_barrier()


def _sc1_body(x_hbm, hist_out, sums_out,
              buf0, buf1, hist, sumbuf, shared, seg2d, accbuf, sem0, sem1):
    cid = lax.axis_index("c")
    sid = lax.axis_index("s")
    wid = sid * _NC + cid
    base = wid * _PER_W
    bufs = (buf0, buf1)
    sems = (sem0, sem1)

    _zero_hist(hist)

    def dma(j, b):
        return pltpu.make_async_copy(
            x_hbm.at[pl.ds(base + j * _CHUNK, _CHUNK)], bufs[b], sems[b])

    dma(0, 0).start()
    dma(1, 1).start()

    ones = jnp.ones((_L,), jnp.int32)

    def process(buf, sums):
        @plsc.parallel_loop(0, _VPC, step=1, unroll=8, carry=sums)
        def vec(i, sums):
            s0, s1 = sums
            xv = buf[pl.ds(i * _L, _L)]
            hb = lax.shift_right_logical(_upattern(xv), 16)
            plsc.addupdate_scatter(hist, [hb], ones)
            return (s1 + xv, s0)

        return vec

    def outer(j2, sums):
        for b in range(2):
            j = j2 * 2 + b
            dma(j, b).wait()
            sums = process(bufs[b], sums)

            @pl.when(j + 2 < _NCHUNKS)
            def _():
                dma(j + 2, b).start()

        return sums

    zero = jnp.zeros((_L,), jnp.float32)
    s0, s1 = lax.fori_loop(0, _NCHUNKS // 2, outer, (zero, zero))
    sumbuf[...] = s0 + s1
    pltpu.sync_copy(sumbuf, sums_out.at[wid])
    _merge_hist(hist, shared, seg2d, accbuf, hist_out, cid, sid)


_sc_pass1 = functools.partial(
    pl.kernel,
    out_type=[
        jax.ShapeDtypeStruct((_NC, _BINS), jnp.int32),
        jax.ShapeDtypeStruct((_NW, _L), jnp.float32),
    ],
    mesh=_mesh,
    compiler_params=_sc_params,
    scratch_types=[
        pltpu.VMEM((_CHUNK,), jnp.float32),
        pltpu.VMEM((_CHUNK,), jnp.float32),
        pltpu.VMEM((_BINS,), jnp.int32),
        pltpu.VMEM((_L,), jnp.float32),
        pltpu.VMEM_SHARED((_NS, _HALF), jnp.int32),
        pltpu.VMEM((8, _SEG), jnp.int32),
        pltpu.VMEM((_SEG,), jnp.int32),
        pltpu.SemaphoreType.DMA,
        pltpu.SemaphoreType.DMA,
    ],
)(_sc1_body)


def _sc2_body(x_hbm, b1_hbm, hist_out, minab_out,
              buf0, buf1, hist, b1buf, minbuf, shared, seg2d, accbuf,
              sem0, sem1):
    cid = lax.axis_index("c")
    sid = lax.axis_index("s")
    wid = sid * _NC + cid
    base = wid * _PER_W
    bufs = (buf0, buf1)
    sems = (sem0, sem1)

    _zero_hist(hist)
    pltpu.sync_copy(b1_hbm, b1buf)
    b1v = b1buf[...]

    def dma(j, b):
        return pltpu.make_async_copy(
            x_hbm.at[pl.ds(base + j * _CHUNK, _CHUNK)], bufs[b], sems[b])

    dma(0, 0).start()
    dma(1, 1).start()

    ones = jnp.ones((_L,), jnp.int32)

    def process(buf, maccs):
        @plsc.parallel_loop(0, _VPC, step=1, unroll=8, carry=maccs)
        def vec(i, maccs):
            m0, m1 = maccs
            xv = buf[pl.ds(i * _L, _L)]
            u = _upattern(xv)
            hb = lax.shift_right_logical(u, 16)
            low = u & 65535
            plsc.addupdate_scatter(hist, [low], ones, mask=hb == b1v)
            key = u ^ jnp.int32(_MIN32)
            cand = jnp.where(hb > b1v, key, jnp.int32(_MAX32))
            return (jnp.minimum(m1, cand), m0)

        return vec

    def outer(j2, maccs):
        for b in range(2):
            j = j2 * 2 + b
            dma(j, b).wait()
            maccs = process(bufs[b], maccs)

            @pl.when(j + 2 < _NCHUNKS)
            def _():
                dma(j + 2, b).start()

        return maccs

    init = jnp.full((_L,), _MAX32, jnp.int32)
    m0, m1 = lax.fori_loop(0, _NCHUNKS // 2, outer, (init, init))
    minbuf[...] = jnp.minimum(m0, m1)
    pltpu.sync_copy(minbuf, minab_out.at[wid])
    _merge_hist(hist, shared, seg2d, accbuf, hist_out, cid, sid)


_sc_pass2 = functools.partial(
    pl.kernel,
    out_type=[
        jax.ShapeDtypeStruct((_NC, _BINS), jnp.int32),
        jax.ShapeDtypeStruct((_NW, _L), jnp.int32),
    ],
    mesh=_mesh,
    compiler_params=_sc_params,
    scratch_types=[
        pltpu.VMEM((_CHUNK,), jnp.float32),
        pltpu.VMEM((_CHUNK,), jnp.float32),
        pltpu.VMEM((_BINS,), jnp.int32),
        pltpu.VMEM((_L,), jnp.int32),
        pltpu.VMEM((_L,), jnp.int32),
        pltpu.VMEM_SHARED((_NS, _HALF), jnp.int32),
        pltpu.VMEM((8, _SEG), jnp.int32),
        pltpu.VMEM((_SEG,), jnp.int32),
        pltpu.SemaphoreType.DMA,
        pltpu.SemaphoreType.DMA,
    ],
)(_sc2_body)


def _descend16(merged, rank):
    idx = (lax.broadcasted_iota(jnp.int32, merged.shape, 0) * 128
           + lax.broadcasted_iota(jnp.int32, merged.shape, 1))

    def step(i, p):
        t = p | (jnp.int32(1) << (jnp.int32(15) - i))
        cnt = jnp.sum(jnp.where(idx < t, merged, 0))
        return jnp.where(cnt > rank, p, t)

    return lax.fori_loop(0, 16, step, jnp.int32(0)), idx


def _an1_body(hist_ref, sums_ref, b1vec_ref, b1s_ref, below_ref, mean_ref):
    merged = hist_ref[0] + hist_ref[1]
    b1, idx = _descend16(merged, jnp.int32(_K))
    below = jnp.sum(jnp.where(idx < b1, merged, 0))

    b1vec_ref[...] = jnp.full((_L,), b1, jnp.int32)
    b1s_ref[0, 0] = b1
    below_ref[0, 0] = below
    mean_ref[0, 0] = jnp.sum(sums_ref[...]) / _N


def _tc_analyze1(hist, sums):
    return pl.pallas_call(
        _an1_body,
        in_specs=[
            pl.BlockSpec((_NC, 512, 128), lambda: (0, 0, 0)),
            pl.BlockSpec((_NW, _L), lambda: (0, 0)),
        ],
        out_specs=[
            pl.BlockSpec((_L,), lambda: (0,)),
            pl.BlockSpec(memory_space=pltpu.SMEM),
            pl.BlockSpec(memory_space=pltpu.SMEM),
            pl.BlockSpec(memory_space=pltpu.SMEM),
        ],
        out_shape=[
            jax.ShapeDtypeStruct((_L,), jnp.int32),
            jax.ShapeDtypeStruct((1, 1), jnp.int32),
            jax.ShapeDtypeStruct((1, 1), jnp.int32),
            jax.ShapeDtypeStruct((1, 1), jnp.float32),
        ],
    )(hist.reshape(_NC, 512, 128), sums)


def _key_to_float(k):
    u = k ^ jnp.int32(_MIN32)
    bits = jnp.where(u < 0, u & jnp.int32(_MAX32), ~u)
    return lax.bitcast_convert_type(bits, jnp.float32)


_MROWS = 8


def _mask_body(hist_ref, minab_ref, b1s_ref, below_ref, mean_ref,
               x_ref, out_ref, q_ref):
    @pl.when(pl.program_id(0) == 0)
    def _():
        b1 = b1s_ref[0, 0]
        below = below_ref[0, 0]

        merged = hist_ref[0] + hist_ref[1]
        cnt_b1 = jnp.sum(merged)
        r = jnp.int32(_K) - below

        low_k, _ = _descend16(merged, r)
        low_k1, _ = _descend16(merged, r + 1)
        min_above = jnp.min(minab_ref[...])

        hi_part = b1 << 16
        key_k = (hi_part | low_k) ^ jnp.int32(_MIN32)
        key_k1 = jnp.where(
            r + 1 < cnt_b1, (hi_part | low_k1) ^ jnp.int32(_MIN32), min_above)

        xk = _key_to_float(key_k)
        xk1 = _key_to_float(key_k1)
        q_ref[0, 0] = xk * (1.0 - _FRAC) + xk1 * _FRAC

    q = q_ref[0, 0]
    m = mean_ref[0, 0]
    xa = x_ref[...]
    out_ref[...] = jnp.where(xa > q, m, xa)


def _tc_mask(image, hist2, minab, b1s, below, meanv):
    return pl.pallas_call(
        _mask_body,
        grid=(_R // _MROWS,),
        in_specs=[
            pl.BlockSpec((_NC, 512, 128), lambda i: (0, 0, 0)),
            pl.BlockSpec((_NW, _L), lambda i: (0, 0)),
            pl.BlockSpec(memory_space=pltpu.SMEM),
            pl.BlockSpec(memory_space=pltpu.SMEM),
            pl.BlockSpec(memory_space=pltpu.SMEM),
            pl.BlockSpec((_MROWS, _C), lambda i: (i, 0)),
        ],
        out_specs=pl.BlockSpec((_MROWS, _C), lambda i: (i, 0)),
        out_shape=jax.ShapeDtypeStruct((_R, _C), jnp.float32),
        scratch_shapes=[pltpu.SMEM((1, 1), jnp.float32)],
    )(hist2.reshape(_NC, 512, 128), minab, b1s, below, meanv, image)


@jax.jit
def kernel(image):
    x1d = image.reshape(_N)
    hist1, sums = _sc_pass1(x1d)
    b1vec, b1s, below, meanv = _tc_analyze1(hist1, sums)
    hist2, minab = _sc_pass2(x1d, b1vec)
    return _tc_mask(image, hist2, minab, b1s, below, meanv)


# speedup vs baseline: 2.8499x; 1.1364x over previous
import functools

import jax
import jax.numpy as jnp
from jax import lax
from jax.experimental import pallas as pl
from jax.experimental.pallas import tpu as pltpu
from jax.experimental.pallas import tpu_sc as plsc

_R, _C = 128, 32768
_N = _R * _C
_POS = 0.75 * (_N - 1)
_K = int(_POS)
_FRAC = _POS - _K
_MIN32 = -2147483648
_MAX32 = 2147483647

_NC, _NS, _L = 2, 16, 16
_NW = _NC * _NS
_PER_W = _N // _NW
_CHUNK = 4096
_NCHUNKS = _PER_W // _CHUNK
_VPC = _CHUNK // _L
_BINS = 65536

_mesh = plsc.VectorSubcoreMesh(core_axis_name="c", subcore_axis_name="s")
_sc_params = pltpu.CompilerParams(needs_layout_passes=False)


def _upattern(xv):
    bits = lax.bitcast_convert_type(xv, jnp.int32)
    return jnp.where(bits >= 0, bits | jnp.int32(_MIN32), ~bits)


def _zero_hist(hist):
    def zero(i, carry):
        hist[pl.ds(i * _L, _L)] = jnp.zeros((_L,), jnp.int32)
        return carry

    lax.fori_loop(0, _BINS // _L, zero, 0, unroll=8)


_HALF = _BINS // 2
_SEG = _HALF // _NS


def _merge_hist(hist, shared, seg2d, accbuf, hist_out, cid, sid):
    for h in range(2):
        off = h * _HALF
        pltpu.sync_copy(hist.at[pl.ds(off, _HALF)], shared.at[sid])
        plsc.subcore_barrier()
        for g in range(2):
            pltpu.sync_copy(
                shared.at[pl.ds(g * 8, 8), pl.ds(sid * _SEG, _SEG)], seg2d)

            @plsc.parallel_loop(0, _SEG // _L, step=1, unroll=4)
            def _(i):
                sl = pl.ds(i * _L, _L)
                v = seg2d[0, sl]
                for t in range(1, 8):
                    v = v + seg2d[t, sl]
                if g == 0:
                    accbuf[sl] = v
                else:
                    accbuf[sl] = accbuf[sl] + v

        pltpu.sync_copy(
            accbuf, hist_out.at[cid, pl.ds(off + sid * _SEG, _SEG)])
        plsc.subcore_barrier()


def _sc1_body(x_hbm, hist_out, sums_out,
              buf0, buf1, hist, sumbuf, shared, seg2d, accbuf, sem0, sem1):
    cid = lax.axis_index("c")
    sid = lax.axis_index("s")
    wid = sid * _NC + cid
    row0 = wid * (_R // _NW)
    bufs = (buf0, buf1)
    sems = (sem0, sem1)

    _zero_hist(hist)

    def dma(j, b):
        r = row0 + (j >> 3)
        c = (j & 7) * _CHUNK
        return pltpu.make_async_copy(
            x_hbm.at[r, pl.ds(c, _CHUNK)], bufs[b], sems[b])

    dma(0, 0).start()
    dma(1, 1).start()

    ones = jnp.ones((_L,), jnp.int32)

    def process(buf, sums):
        @plsc.parallel_loop(0, _VPC, step=1, unroll=8, carry=sums)
        def vec(i, sums):
            s0, s1 = sums
            xv = buf[pl.ds(i * _L, _L)]
            hb = lax.shift_right_logical(_upattern(xv), 16)
            plsc.addupdate_scatter(hist, [hb], ones)
            return (s1 + xv, s0)

        return vec

    def outer(j2, sums):
        for b in range(2):
            j = j2 * 2 + b
            dma(j, b).wait()
            sums = process(bufs[b], sums)

            @pl.when(j + 2 < _NCHUNKS)
            def _():
                dma(j + 2, b).start()

        return sums

    zero = jnp.zeros((_L,), jnp.float32)
    s0, s1 = lax.fori_loop(0, _NCHUNKS // 2, outer, (zero, zero))
    sumbuf[...] = s0 + s1
    pltpu.sync_copy(sumbuf, sums_out.at[wid])
    _merge_hist(hist, shared, seg2d, accbuf, hist_out, cid, sid)


_sc_pass1 = functools.partial(
    pl.kernel,
    out_type=[
        jax.ShapeDtypeStruct((_NC, _BINS), jnp.int32),
        jax.ShapeDtypeStruct((_NW, _L), jnp.float32),
    ],
    mesh=_mesh,
    compiler_params=_sc_params,
    scratch_types=[
        pltpu.VMEM((_CHUNK,), jnp.float32),
        pltpu.VMEM((_CHUNK,), jnp.float32),
        pltpu.VMEM((_BINS,), jnp.int32),
        pltpu.VMEM((_L,), jnp.float32),
        pltpu.VMEM_SHARED((_NS, _HALF), jnp.int32),
        pltpu.VMEM((8, _SEG), jnp.int32),
        pltpu.VMEM((_SEG,), jnp.int32),
        pltpu.SemaphoreType.DMA,
        pltpu.SemaphoreType.DMA,
    ],
)(_sc1_body)


def _sc2_body(x_hbm, b1_hbm, hist_out, minab_out,
              buf0, buf1, hist, b1buf, minbuf, shared, seg2d, accbuf,
              sem0, sem1):
    cid = lax.axis_index("c")
    sid = lax.axis_index("s")
    wid = sid * _NC + cid
    row0 = wid * (_R // _NW)
    bufs = (buf0, buf1)
    sems = (sem0, sem1)

    _zero_hist(hist)
    pltpu.sync_copy(b1_hbm, b1buf)
    b1v = b1buf[...]

    def dma(j, b):
        r = row0 + (j >> 3)
        c = (j & 7) * _CHUNK
        return pltpu.make_async_copy(
            x_hbm.at[r, pl.ds(c, _CHUNK)], bufs[b], sems[b])

    dma(0, 0).start()
    dma(1, 1).start()

    ones = jnp.ones((_L,), jnp.int32)

    def process(buf, maccs):
        @plsc.parallel_loop(0, _VPC, step=1, unroll=8, carry=maccs)
        def vec(i, maccs):
            m0, m1 = maccs
            xv = buf[pl.ds(i * _L, _L)]
            u = _upattern(xv)
            hb = lax.shift_right_logical(u, 16)
            low = u & 65535
            plsc.addupdate_scatter(hist, [low], ones, mask=hb == b1v)
            key = u ^ jnp.int32(_MIN32)
            cand = jnp.where(hb > b1v, key, jnp.int32(_MAX32))
            return (jnp.minimum(m1, cand), m0)

        return vec

    def outer(j2, maccs):
        for b in range(2):
            j = j2 * 2 + b
            dma(j, b).wait()
            maccs = process(bufs[b], maccs)

            @pl.when(j + 2 < _NCHUNKS)
            def _():
                dma(j + 2, b).start()

        return maccs

    init = jnp.full((_L,), _MAX32, jnp.int32)
    m0, m1 = lax.fori_loop(0, _NCHUNKS // 2, outer, (init, init))
    minbuf[...] = jnp.minimum(m0, m1)
    pltpu.sync_copy(minbuf, minab_out.at[wid])
    _merge_hist(hist, shared, seg2d, accbuf, hist_out, cid, sid)


_sc_pass2 = functools.partial(
    pl.kernel,
    out_type=[
        jax.ShapeDtypeStruct((_NC, _BINS), jnp.int32),
        jax.ShapeDtypeStruct((_NW, _L), jnp.int32),
    ],
    mesh=_mesh,
    compiler_params=_sc_params,
    scratch_types=[
        pltpu.VMEM((_CHUNK,), jnp.float32),
        pltpu.VMEM((_CHUNK,), jnp.float32),
        pltpu.VMEM((_BINS,), jnp.int32),
        pltpu.VMEM((_L,), jnp.int32),
        pltpu.VMEM((_L,), jnp.int32),
        pltpu.VMEM_SHARED((_NS, _HALF), jnp.int32),
        pltpu.VMEM((8, _SEG), jnp.int32),
        pltpu.VMEM((_SEG,), jnp.int32),
        pltpu.SemaphoreType.DMA,
        pltpu.SemaphoreType.DMA,
    ],
)(_sc2_body)


def _descend16(merged, rank):
    idx = (lax.broadcasted_iota(jnp.int32, merged.shape, 0) * 128
           + lax.broadcasted_iota(jnp.int32, merged.shape, 1))

    def step(i, p):
        t = p | (jnp.int32(1) << (jnp.int32(15) - i))
        cnt = jnp.sum(jnp.where(idx < t, merged, 0))
        return jnp.where(cnt > rank, p, t)

    return lax.fori_loop(0, 16, step, jnp.int32(0)), idx


def _an1_body(hist_ref, sums_ref, b1vec_ref, b1s_ref, below_ref, mean_ref):
    merged = hist_ref[0] + hist_ref[1]
    b1, idx = _descend16(merged, jnp.int32(_K))
    below = jnp.sum(jnp.where(idx < b1, merged, 0))

    b1vec_ref[...] = jnp.full((_L,), b1, jnp.int32)
    b1s_ref[0, 0] = b1
    below_ref[0, 0] = below
    mean_ref[0, 0] = jnp.sum(sums_ref[...]) / _N


def _tc_analyze1(hist, sums):
    return pl.pallas_call(
        _an1_body,
        in_specs=[
            pl.BlockSpec((_NC, 512, 128), lambda: (0, 0, 0)),
            pl.BlockSpec((_NW, _L), lambda: (0, 0)),
        ],
        out_specs=[
            pl.BlockSpec((_L,), lambda: (0,)),
            pl.BlockSpec(memory_space=pltpu.SMEM),
            pl.BlockSpec(memory_space=pltpu.SMEM),
            pl.BlockSpec(memory_space=pltpu.SMEM),
        ],
        out_shape=[
            jax.ShapeDtypeStruct((_L,), jnp.int32),
            jax.ShapeDtypeStruct((1, 1), jnp.int32),
            jax.ShapeDtypeStruct((1, 1), jnp.int32),
            jax.ShapeDtypeStruct((1, 1), jnp.float32),
        ],
    )(hist.reshape(_NC, 512, 128), sums)


def _key_to_float(k):
    u = k ^ jnp.int32(_MIN32)
    bits = jnp.where(u < 0, u & jnp.int32(_MAX32), ~u)
    return lax.bitcast_convert_type(bits, jnp.float32)


_MROWS = 8


def _mask_body(hist_ref, minab_ref, b1s_ref, below_ref, mean_ref,
               x_ref, out_ref, q_ref):
    @pl.when(pl.program_id(0) == 0)
    def _():
        b1 = b1s_ref[0, 0]
        below = below_ref[0, 0]

        merged = hist_ref[0] + hist_ref[1]
        cnt_b1 = jnp.sum(merged)
        r = jnp.int32(_K) - below

        low_k, _ = _descend16(merged, r)
        low_k1, _ = _descend16(merged, r + 1)
        min_above = jnp.min(minab_ref[...])

        hi_part = b1 << 16
        key_k = (hi_part | low_k) ^ jnp.int32(_MIN32)
        key_k1 = jnp.where(
            r + 1 < cnt_b1, (hi_part | low_k1) ^ jnp.int32(_MIN32), min_above)

        xk = _key_to_float(key_k)
        xk1 = _key_to_float(key_k1)
        q_ref[0, 0] = xk * (1.0 - _FRAC) + xk1 * _FRAC

    q = q_ref[0, 0]
    m = mean_ref[0, 0]
    xa = x_ref[...]
    out_ref[...] = jnp.where(xa > q, m, xa)


def _tc_mask(image, hist2, minab, b1s, below, meanv):
    return pl.pallas_call(
        _mask_body,
        grid=(_R // _MROWS,),
        in_specs=[
            pl.BlockSpec((_NC, 512, 128), lambda i: (0, 0, 0)),
            pl.BlockSpec((_NW, _L), lambda i: (0, 0)),
            pl.BlockSpec(memory_space=pltpu.SMEM),
            pl.BlockSpec(memory_space=pltpu.SMEM),
            pl.BlockSpec(memory_space=pltpu.SMEM),
            pl.BlockSpec((_MROWS, _C), lambda i: (i, 0)),
        ],
        out_specs=pl.BlockSpec((_MROWS, _C), lambda i: (i, 0)),
        out_shape=jax.ShapeDtypeStruct((_R, _C), jnp.float32),
        scratch_shapes=[pltpu.SMEM((1, 1), jnp.float32)],
    )(hist2.reshape(_NC, 512, 128), minab, b1s, below, meanv, image)


@jax.jit
def kernel(image):
    hist1, sums = _sc_pass1(image)
    b1vec, b1s, below, meanv = _tc_analyze1(hist1, sums)
    hist2, minab = _sc_pass2(image, b1vec)
    return _tc_mask(image, hist2, minab, b1s, below, meanv)
